# baseline (speedup 1.0000x reference)
import functools

import jax
import jax.numpy as jnp
from jax import lax
from jax.experimental import pallas as pl
from jax.experimental.pallas import tpu as pltpu
from jax.experimental.pallas import tpu_sc as plsc

N = 6_400_000
S_SEG = 10_000
L = 16
NW = 32
SPAD = 10_240
COLS = SPAD // NW
CHUNK = N // NW
BLK = 8_000
NBLK = CHUNK // BLK
SUBL = BLK // L
UNROLL = 4

_MESH = dict(core_axis_name="c", subcore_axis_name="s")
_PARAMS = pltpu.CompilerParams(
    needs_layout_passes=False, use_tc_tiling_on_sc=False
)


def _take(x, idx):
    return jnp.take_along_axis(x, idx, axis=0)


def _phase1(data, ids):
    mesh = plsc.VectorSubcoreMesh(**_MESH)

    @functools.partial(
        pl.kernel,
        out_type=jax.ShapeDtypeStruct((NW, SPAD), jnp.float32),
        mesh=mesh,
        scratch_types=[
            pltpu.VMEM((2, BLK), jnp.float32),
            pltpu.VMEM((2, BLK), jnp.int32),
            pltpu.VMEM((SPAD,), jnp.float32),
            pltpu.VMEM((NBLK * L,), jnp.int32),
            pltpu.VMEM((NBLK * L,), jnp.float32),
            pltpu.SemaphoreType.DMA,
            pltpu.SemaphoreType.DMA,
            pltpu.SemaphoreType.DMA,
            pltpu.SemaphoreType.DMA,
            pltpu.SemaphoreType.DMA,
        ],
        compiler_params=_PARAMS,
    )
    def k(data_hbm, ids_hbm, part_hbm, dbuf, ibuf, acc, lid, lval, sd0, sd1,
          si0, si1, so):
        wid = lax.axis_index("c") * 16 + lax.axis_index("s")
        base = wid * CHUNK
        dsem = (sd0, sd1)
        isem = (si0, si1)

        minf = jnp.full((L,), -jnp.inf, dtype=jnp.float32)
        iota = lax.iota(jnp.int32, L)
        last = jnp.full((L,), L - 1, dtype=jnp.int32)

        def ibody(i, c):
            acc[pl.ds(i * L, L)] = minf
            return c

        lax.fori_loop(0, SPAD // L, ibody, 0)

        def issue(b):
            slot = b % 2
            off = base + b * BLK
            pltpu.async_copy(data_hbm.at[pl.ds(off, BLK)], dbuf.at[slot],
                             dsem[slot])
            pltpu.async_copy(ids_hbm.at[pl.ds(off, BLK)], ibuf.at[slot],
                             isem[slot])

        def wait(b):
            slot = b % 2
            off = base + b * BLK
            pltpu.make_async_copy(data_hbm.at[pl.ds(off, BLK)], dbuf.at[slot],
                                  dsem[slot]).wait()
            pltpu.make_async_copy(ids_hbm.at[pl.ds(off, BLK)], ibuf.at[slot],
                                  isem[slot]).wait()

        lane_base = iota * SUBL

        def process(slot, b):
            prev = plsc.load_gather(ibuf.at[slot], [lane_base])
            accv = minf

            def sbody(k, carry):
                prev, accv = carry
                for u in range(UNROLL):
                    fidx = lane_base + (k * UNROLL + u)
                    idv = plsc.load_gather(ibuf.at[slot], [fidx])
                    dv = plsc.load_gather(dbuf.at[slot], [fidx])
                    changed = idv != prev
                    plsc.store_scatter(acc, [prev], accv, mask=changed)
                    accv = jnp.where(changed, dv, jnp.maximum(accv, dv))
                    prev = idv
                return prev, accv

            prev, accv = lax.fori_loop(0, SUBL // UNROLL, sbody,
                                       (prev, accv))
            lid[pl.ds(b * L, L)] = prev
            lval[pl.ds(b * L, L)] = accv

        issue(0)
        for b in range(NBLK):
            if b + 1 < NBLK:
                issue(b + 1)
            wait(b)
            process(b % 2, b)

        def lbody(e, c):
            seg = lid[pl.ds(e * L, L)]
            vals = lval[pl.ds(e * L, L)]
            for sh in (1, 2, 4, 8):
                pidx = jnp.maximum(iota - sh, 0)
                gseg = _take(seg, pidx)
                gval = _take(vals, pidx)
                vals = jnp.where(seg == gseg, jnp.maximum(vals, gval), vals)
            nseg = _take(seg, jnp.minimum(iota + 1, last))
            end = (seg != nseg) | (iota == last)
            cur = plsc.load_gather(acc, [seg])
            plsc.store_scatter(acc, [seg], jnp.maximum(cur, vals), mask=end)
            return c

        lax.fori_loop(0, NBLK, lbody, 0)

        pltpu.async_copy(acc, part_hbm.at[wid], so).wait()

    return k(data, ids)


def _phase2(part):
    mesh = plsc.VectorSubcoreMesh(**_MESH)

    @functools.partial(
        pl.kernel,
        out_type=jax.ShapeDtypeStruct((SPAD,), jnp.float32),
        mesh=mesh,
        scratch_types=[
            pltpu.VMEM((NW, COLS), jnp.float32),
            pltpu.VMEM((COLS,), jnp.float32),
            pltpu.SemaphoreType.DMA,
        ],
        compiler_params=_PARAMS,
    )
    def k(part_hbm, out_hbm, buf, obuf, sem):
        wid = lax.axis_index("c") * 16 + lax.axis_index("s")
        col0 = wid * COLS
        for r in range(NW):
            pltpu.async_copy(part_hbm.at[r, pl.ds(col0, COLS)], buf.at[r],
                             sem)
        for r in range(NW):
            pltpu.make_async_copy(part_hbm.at[r, pl.ds(col0, COLS)],
                                  buf.at[r], sem).wait()

        def cbody(j, c):
            off = j * L
            m = buf[0, pl.ds(off, L)]
            for r in range(1, NW):
                m = jnp.maximum(m, buf[r, pl.ds(off, L)])
            obuf[pl.ds(off, L)] = m
            return c

        lax.fori_loop(0, COLS // L, cbody, 0)
        pltpu.async_copy(obuf, out_hbm.at[pl.ds(col0, COLS)], sem).wait()

    return k(part)


def kernel(data, segment_ids, num_segments):
    del num_segments
    ids = segment_ids.astype(jnp.int32)
    part = _phase1(data, ids)
    out = _phase2(part)
    return out[:S_SEG]


# speedup vs baseline: 2.4290x; 2.4290x over previous
import functools

import jax
import jax.numpy as jnp
from jax import lax
from jax.experimental import pallas as pl
from jax.experimental.pallas import tpu as pltpu
from jax.experimental.pallas import tpu_sc as plsc

N = 6_400_000
S_SEG = 10_000
L = 16
NW = 32
SPAD = 10_240
COLS = SPAD // NW
CHUNK = N // NW
BLK = 8_000
NBLK = CHUNK // BLK
SUBL = BLK // L
UNROLL = 4

_MESH = dict(core_axis_name="c", subcore_axis_name="s")
_PARAMS = pltpu.CompilerParams(
    needs_layout_passes=False, use_tc_tiling_on_sc=False
)


def _take(x, idx):
    return jnp.take_along_axis(x, idx, axis=0)


def _phase1(data, ids):
    mesh = plsc.VectorSubcoreMesh(**_MESH)

    @functools.partial(
        pl.kernel,
        out_type=jax.ShapeDtypeStruct((NW, SPAD), jnp.float32),
        mesh=mesh,
        scratch_types=[
            pltpu.VMEM((2, BLK), jnp.float32),
            pltpu.VMEM((2, BLK), jnp.int32),
            pltpu.VMEM((SPAD,), jnp.float32),
            pltpu.VMEM((NBLK * L,), jnp.int32),
            pltpu.VMEM((NBLK * L,), jnp.float32),
            pltpu.SemaphoreType.DMA,
            pltpu.SemaphoreType.DMA,
            pltpu.SemaphoreType.DMA,
            pltpu.SemaphoreType.DMA,
            pltpu.SemaphoreType.DMA,
        ],
        compiler_params=_PARAMS,
    )
    def k(data_hbm, ids_hbm, part_hbm, dbuf, ibuf, acc, lid, lval, sd0, sd1,
          si0, si1, so):
        wid = lax.axis_index("c") * 16 + lax.axis_index("s")
        base = wid * CHUNK
        dsem = (sd0, sd1)
        isem = (si0, si1)

        minf = jnp.full((L,), -jnp.inf, dtype=jnp.float32)
        iota = lax.iota(jnp.int32, L)
        last = jnp.full((L,), L - 1, dtype=jnp.int32)

        def ibody(i, c):
            acc[pl.ds(i * L, L)] = minf
            return c

        lax.fori_loop(0, SPAD // L, ibody, 0)

        def issue(b):
            slot = b % 2
            off = base + b * BLK
            pltpu.async_copy(data_hbm.at[pl.ds(off, BLK)], dbuf.at[slot],
                             dsem[slot])
            pltpu.async_copy(ids_hbm.at[pl.ds(off, BLK)], ibuf.at[slot],
                             isem[slot])

        def wait(b):
            slot = b % 2
            off = base + b * BLK
            pltpu.make_async_copy(data_hbm.at[pl.ds(off, BLK)], dbuf.at[slot],
                                  dsem[slot]).wait()
            pltpu.make_async_copy(ids_hbm.at[pl.ds(off, BLK)], ibuf.at[slot],
                                  isem[slot]).wait()

        lane_base = iota * SUBL

        def process(slot, b):
            prev0 = plsc.load_gather(ibuf.at[slot], [lane_base])

            @plsc.parallel_loop(0, SUBL, unroll=UNROLL,
                                carry=(prev0, minf))
            def step(t, carry):
                prev, accv = carry
                fidx = lane_base + t
                idv = plsc.load_gather(ibuf.at[slot], [fidx])
                dv = plsc.load_gather(dbuf.at[slot], [fidx])
                changed = idv != prev
                plsc.store_scatter(acc, [prev], accv, mask=changed)
                accv = jnp.where(changed, dv, jnp.maximum(accv, dv))
                return idv, accv

            prev, accv = step
            lid[pl.ds(b * L, L)] = prev
            lval[pl.ds(b * L, L)] = accv

        issue(0)
        for b in range(NBLK):
            if b + 1 < NBLK:
                issue(b + 1)
            wait(b)
            process(b % 2, b)

        def lbody(e, c):
            seg = lid[pl.ds(e * L, L)]
            vals = lval[pl.ds(e * L, L)]
            for sh in (1, 2, 4, 8):
                pidx = jnp.maximum(iota - sh, 0)
                gseg = _take(seg, pidx)
                gval = _take(vals, pidx)
                vals = jnp.where(seg == gseg, jnp.maximum(vals, gval), vals)
            nseg = _take(seg, jnp.minimum(iota + 1, last))
            end = (seg != nseg) | (iota == last)
            cur = plsc.load_gather(acc, [seg])
            plsc.store_scatter(acc, [seg], jnp.maximum(cur, vals), mask=end)
            return c

        lax.fori_loop(0, NBLK, lbody, 0)

        pltpu.async_copy(acc, part_hbm.at[wid], so).wait()

    return k(data, ids)


def _phase2(part):
    mesh = plsc.VectorSubcoreMesh(**_MESH)

    @functools.partial(
        pl.kernel,
        out_type=jax.ShapeDtypeStruct((SPAD,), jnp.float32),
        mesh=mesh,
        scratch_types=[
            pltpu.VMEM((NW, COLS), jnp.float32),
            pltpu.VMEM((COLS,), jnp.float32),
            pltpu.SemaphoreType.DMA,
        ],
        compiler_params=_PARAMS,
    )
    def k(part_hbm, out_hbm, buf, obuf, sem):
        wid = lax.axis_index("c") * 16 + lax.axis_index("s")
        col0 = wid * COLS
        for r in range(NW):
            pltpu.async_copy(part_hbm.at[r, pl.ds(col0, COLS)], buf.at[r],
                             sem)
        for r in range(NW):
            pltpu.make_async_copy(part_hbm.at[r, pl.ds(col0, COLS)],
                                  buf.at[r], sem).wait()

        def cbody(j, c):
            off = j * L
            m = buf[0, pl.ds(off, L)]
            for r in range(1, NW):
                m = jnp.maximum(m, buf[r, pl.ds(off, L)])
            obuf[pl.ds(off, L)] = m
            return c

        lax.fori_loop(0, COLS // L, cbody, 0)
        pltpu.async_copy(obuf, out_hbm.at[pl.ds(col0, COLS)], sem).wait()

    return k(part)


def kernel(data, segment_ids, num_segments):
    del num_segments
    ids = segment_ids.astype(jnp.int32)
    part = _phase1(data, ids)
    out = _phase2(part)
    return out[:S_SEG]


# speedup vs baseline: 2.4637x; 1.0143x over previous
import functools

import jax
import jax.numpy as jnp
from jax import lax
from jax.experimental import pallas as pl
from jax.experimental.pallas import tpu as pltpu
from jax.experimental.pallas import tpu_sc as plsc

N = 6_400_000
S_SEG = 10_000
L = 16
NW = 32
SPAD = 10_240
COLS = SPAD // NW
CHUNK = N // NW
BLK = 8_000
NBLK = CHUNK // BLK
SUBL = BLK // L
UNROLL = 4

_MESH = dict(core_axis_name="c", subcore_axis_name="s")
_PARAMS = pltpu.CompilerParams(
    needs_layout_passes=False, use_tc_tiling_on_sc=False
)


def _take(x, idx):
    return jnp.take_along_axis(x, idx, axis=0)


def _phase1(data, ids):
    mesh = plsc.VectorSubcoreMesh(**_MESH)

    @functools.partial(
        pl.kernel,
        out_type=jax.ShapeDtypeStruct((NW, SPAD), jnp.float32),
        mesh=mesh,
        scratch_types=[
            pltpu.VMEM((2, BLK), jnp.float32),
            pltpu.VMEM((2, BLK), jnp.int32),
            pltpu.VMEM((SPAD,), jnp.float32),
            pltpu.VMEM((NBLK * L,), jnp.int32),
            pltpu.VMEM((NBLK * L,), jnp.float32),
            pltpu.SemaphoreType.DMA,
            pltpu.SemaphoreType.DMA,
            pltpu.SemaphoreType.DMA,
            pltpu.SemaphoreType.DMA,
            pltpu.SemaphoreType.DMA,
        ],
        compiler_params=_PARAMS,
    )
    def k(data_hbm, ids_hbm, part_hbm, dbuf, ibuf, acc, lid, lval, sd0, sd1,
          si0, si1, so):
        wid = lax.axis_index("c") * 16 + lax.axis_index("s")
        base = wid * CHUNK
        dsem = (sd0, sd1)
        isem = (si0, si1)

        minf = jnp.full((L,), -jnp.inf, dtype=jnp.float32)
        iota = lax.iota(jnp.int32, L)
        last = jnp.full((L,), L - 1, dtype=jnp.int32)

        def ibody(i, c):
            acc[pl.ds(i * L, L)] = minf
            return c

        lax.fori_loop(0, SPAD // L, ibody, 0)

        def issue(b):
            slot = b % 2
            off = base + b * BLK
            pltpu.async_copy(data_hbm.at[pl.ds(off, BLK)], dbuf.at[slot],
                             dsem[slot])
            pltpu.async_copy(ids_hbm.at[pl.ds(off, BLK)], ibuf.at[slot],
                             isem[slot])

        def wait(b):
            slot = b % 2
            off = base + b * BLK
            pltpu.make_async_copy(data_hbm.at[pl.ds(off, BLK)], dbuf.at[slot],
                                  dsem[slot]).wait()
            pltpu.make_async_copy(ids_hbm.at[pl.ds(off, BLK)], ibuf.at[slot],
                                  isem[slot]).wait()

        lane_base = iota * SUBL

        def process(slot, b):
            prev0 = plsc.load_gather(ibuf.at[slot], [lane_base])

            @plsc.parallel_loop(0, SUBL, unroll=UNROLL,
                                carry=(prev0, minf))
            def step(t, carry):
                prev, accv = carry
                fidx = lane_base + t
                idv = plsc.load_gather(ibuf.at[slot], [fidx])
                dv = plsc.load_gather(dbuf.at[slot], [fidx])
                changed = idv != prev
                plsc.store_scatter(acc, [prev], accv, mask=changed)
                accv = jnp.where(changed, dv, jnp.maximum(accv, dv))
                return idv, accv

            prev, accv = step
            lid[pl.ds(b * L, L)] = prev
            lval[pl.ds(b * L, L)] = accv

        issue(0)
        for b in range(NBLK):
            if b + 1 < NBLK:
                issue(b + 1)
            wait(b)
            process(b % 2, b)

        def lbody(e, c):
            seg = lid[pl.ds(e * L, L)]
            vals = lval[pl.ds(e * L, L)]
            for sh in (1, 2, 4, 8):
                pidx = jnp.maximum(iota - sh, 0)
                gseg = _take(seg, pidx)
                gval = _take(vals, pidx)
                vals = jnp.where(seg == gseg, jnp.maximum(vals, gval), vals)
            nseg = _take(seg, jnp.minimum(iota + 1, last))
            end = (seg != nseg) | (iota == last)
            cur = plsc.load_gather(acc, [seg])
            plsc.store_scatter(acc, [seg], jnp.maximum(cur, vals), mask=end)
            return c

        lax.fori_loop(0, NBLK, lbody, 0)

        pltpu.async_copy(acc, part_hbm.at[wid], so).wait()

    return k(data, ids)


def _phase2(part):
    def k(part_ref, out_ref):
        out_ref[...] = jnp.max(part_ref[...], axis=0)

    return pl.pallas_call(
        k,
        out_shape=jax.ShapeDtypeStruct((SPAD,), jnp.float32),
    )(part)


def kernel(data, segment_ids, num_segments):
    del num_segments
    ids = segment_ids.astype(jnp.int32)
    part = _phase1(data, ids)
    out = _phase2(part)
    return out[:S_SEG]


# speedup vs baseline: 2.7375x; 1.1111x over previous
import functools

import jax
import jax.numpy as jnp
from jax import lax
from jax.experimental import pallas as pl
from jax.experimental.pallas import tpu as pltpu
from jax.experimental.pallas import tpu_sc as plsc

N = 6_400_000
S_SEG = 10_000
L = 16
NW = 32
SPAD = 10_240
COLS = SPAD // NW
CHUNK = N // NW
BLK = 20_000
NBLK = CHUNK // BLK
SUBL = BLK // L
UNROLL = 5

_MESH = dict(core_axis_name="c", subcore_axis_name="s")
_PARAMS = pltpu.CompilerParams(
    needs_layout_passes=False, use_tc_tiling_on_sc=False
)


def _take(x, idx):
    return jnp.take_along_axis(x, idx, axis=0)


def _phase1(data, ids):
    mesh = plsc.VectorSubcoreMesh(**_MESH)

    @functools.partial(
        pl.kernel,
        out_type=jax.ShapeDtypeStruct((NW, SPAD), jnp.float32),
        mesh=mesh,
        scratch_types=[
            pltpu.VMEM((2, BLK), jnp.float32),
            pltpu.VMEM((2, BLK), jnp.int32),
            pltpu.VMEM((SPAD,), jnp.float32),
            pltpu.VMEM((NBLK * L,), jnp.int32),
            pltpu.VMEM((NBLK * L,), jnp.float32),
            pltpu.SemaphoreType.DMA,
            pltpu.SemaphoreType.DMA,
            pltpu.SemaphoreType.DMA,
            pltpu.SemaphoreType.DMA,
            pltpu.SemaphoreType.DMA,
        ],
        compiler_params=_PARAMS,
    )
    def k(data_hbm, ids_hbm, part_hbm, dbuf, ibuf, acc, lid, lval, sd0, sd1,
          si0, si1, so):
        wid = lax.axis_index("c") * 16 + lax.axis_index("s")
        base = wid * CHUNK
        dsem = (sd0, sd1)
        isem = (si0, si1)

        minf = jnp.full((L,), -jnp.inf, dtype=jnp.float32)
        iota = lax.iota(jnp.int32, L)
        last = jnp.full((L,), L - 1, dtype=jnp.int32)

        def ibody(i, c):
            acc[pl.ds(i * L, L)] = minf
            return c

        lax.fori_loop(0, SPAD // L, ibody, 0)

        def issue(b):
            slot = b % 2
            off = base + b * BLK
            pltpu.async_copy(data_hbm.at[pl.ds(off, BLK)], dbuf.at[slot],
                             dsem[slot])
            pltpu.async_copy(ids_hbm.at[pl.ds(off, BLK)], ibuf.at[slot],
                             isem[slot])

        def wait(b):
            slot = b % 2
            off = base + b * BLK
            pltpu.make_async_copy(data_hbm.at[pl.ds(off, BLK)], dbuf.at[slot],
                                  dsem[slot]).wait()
            pltpu.make_async_copy(ids_hbm.at[pl.ds(off, BLK)], ibuf.at[slot],
                                  isem[slot]).wait()

        lane_base = iota * SUBL

        def process(slot, b):
            prev0 = plsc.load_gather(ibuf.at[slot], [lane_base])

            @plsc.parallel_loop(0, SUBL, unroll=UNROLL,
                                carry=(prev0, minf))
            def step(t, carry):
                prev, accv = carry
                fidx = lane_base + t
                idv = plsc.load_gather(ibuf.at[slot], [fidx])
                dv = plsc.load_gather(dbuf.at[slot], [fidx])
                changed = idv != prev
                plsc.store_scatter(acc, [prev], accv, mask=changed)
                accv = jnp.where(changed, dv, jnp.maximum(accv, dv))
                return idv, accv

            prev, accv = step
            lid[pl.ds(b * L, L)] = prev
            lval[pl.ds(b * L, L)] = accv

        issue(0)
        for b in range(NBLK):
            if b + 1 < NBLK:
                issue(b + 1)
            wait(b)
            process(b % 2, b)

        def lbody(e, c):
            seg = lid[pl.ds(e * L, L)]
            vals = lval[pl.ds(e * L, L)]
            for sh in (1, 2, 4, 8):
                pidx = jnp.maximum(iota - sh, 0)
                gseg = _take(seg, pidx)
                gval = _take(vals, pidx)
                vals = jnp.where(seg == gseg, jnp.maximum(vals, gval), vals)
            nseg = _take(seg, jnp.minimum(iota + 1, last))
            end = (seg != nseg) | (iota == last)
            cur = plsc.load_gather(acc, [seg])
            plsc.store_scatter(acc, [seg], jnp.maximum(cur, vals), mask=end)
            return c

        lax.fori_loop(0, NBLK, lbody, 0)

        pltpu.async_copy(acc, part_hbm.at[wid], so).wait()

    return k(data, ids)


def _phase2(part):
    def k(part_ref, out_ref):
        out_ref[...] = jnp.max(part_ref[...], axis=0)

    return pl.pallas_call(
        k,
        out_shape=jax.ShapeDtypeStruct((SPAD,), jnp.float32),
    )(part)


def kernel(data, segment_ids, num_segments):
    del num_segments
    ids = segment_ids.astype(jnp.int32)
    part = _phase1(data, ids)
    out = _phase2(part)
    return out[:S_SEG]


# speedup vs baseline: 2.7411x; 1.0013x over previous
import functools

import jax
import jax.numpy as jnp
from jax import lax
from jax.experimental import pallas as pl
from jax.experimental.pallas import tpu as pltpu
from jax.experimental.pallas import tpu_sc as plsc

N = 6_400_000
S_SEG = 10_000
L = 16
NW = 32
SPAD = 10_240
COLS = SPAD // NW
CHUNK = N // NW
BLK = 20_000
NBLK = CHUNK // BLK
SUBL = BLK // L
UNROLL = 5

_MESH = dict(core_axis_name="c", subcore_axis_name="s")
_PARAMS = pltpu.CompilerParams(
    needs_layout_passes=False, use_tc_tiling_on_sc=False,
    skip_device_barrier=True,
)


def _take(x, idx):
    return jnp.take_along_axis(x, idx, axis=0)


def _phase1(data, ids):
    mesh = plsc.VectorSubcoreMesh(**_MESH)

    @functools.partial(
        pl.kernel,
        out_type=jax.ShapeDtypeStruct((NW, SPAD), jnp.float32),
        mesh=mesh,
        scratch_types=[
            pltpu.VMEM((2, BLK), jnp.float32),
            pltpu.VMEM((2, BLK), jnp.int32),
            pltpu.VMEM((SPAD,), jnp.float32),
            pltpu.VMEM((NBLK * L,), jnp.int32),
            pltpu.VMEM((NBLK * L,), jnp.float32),
            pltpu.SemaphoreType.DMA,
            pltpu.SemaphoreType.DMA,
            pltpu.SemaphoreType.DMA,
            pltpu.SemaphoreType.DMA,
            pltpu.SemaphoreType.DMA,
        ],
        compiler_params=_PARAMS,
    )
    def k(data_hbm, ids_hbm, part_hbm, dbuf, ibuf, acc, lid, lval, sd0, sd1,
          si0, si1, so):
        wid = lax.axis_index("c") * 16 + lax.axis_index("s")
        base = wid * CHUNK
        dsem = (sd0, sd1)
        isem = (si0, si1)

        minf = jnp.full((L,), -jnp.inf, dtype=jnp.float32)
        iota = lax.iota(jnp.int32, L)
        last = jnp.full((L,), L - 1, dtype=jnp.int32)

        def ibody(i, c):
            acc[pl.ds(i * L, L)] = minf
            return c

        lax.fori_loop(0, SPAD // L, ibody, 0)

        def issue(b):
            slot = b % 2
            off = base + b * BLK
            pltpu.async_copy(data_hbm.at[pl.ds(off, BLK)], dbuf.at[slot],
                             dsem[slot])
            pltpu.async_copy(ids_hbm.at[pl.ds(off, BLK)], ibuf.at[slot],
                             isem[slot])

        def wait(b):
            slot = b % 2
            off = base + b * BLK
            pltpu.make_async_copy(data_hbm.at[pl.ds(off, BLK)], dbuf.at[slot],
                                  dsem[slot]).wait()
            pltpu.make_async_copy(ids_hbm.at[pl.ds(off, BLK)], ibuf.at[slot],
                                  isem[slot]).wait()

        lane_base = iota * SUBL

        def process(slot, b):
            prev0 = plsc.load_gather(ibuf.at[slot], [lane_base])

            @plsc.parallel_loop(0, SUBL, unroll=UNROLL,
                                carry=(prev0, minf))
            def step(t, carry):
                prev, accv = carry
                fidx = lane_base + t
                idv = plsc.load_gather(ibuf.at[slot], [fidx])
                dv = plsc.load_gather(dbuf.at[slot], [fidx])
                changed = idv != prev
                plsc.store_scatter(acc, [prev], accv, mask=changed)
                accv = jnp.where(changed, dv, jnp.maximum(accv, dv))
                return idv, accv

            prev, accv = step
            lid[pl.ds(b * L, L)] = prev
            lval[pl.ds(b * L, L)] = accv

        issue(0)
        for b in range(NBLK):
            if b + 1 < NBLK:
                issue(b + 1)
            wait(b)
            process(b % 2, b)

        def lbody(e, c):
            seg = lid[pl.ds(e * L, L)]
            vals = lval[pl.ds(e * L, L)]
            for sh in (1, 2, 4, 8):
                pidx = jnp.maximum(iota - sh, 0)
                gseg = _take(seg, pidx)
                gval = _take(vals, pidx)
                vals = jnp.where(seg == gseg, jnp.maximum(vals, gval), vals)
            nseg = _take(seg, jnp.minimum(iota + 1, last))
            end = (seg != nseg) | (iota == last)
            cur = plsc.load_gather(acc, [seg])
            plsc.store_scatter(acc, [seg], jnp.maximum(cur, vals), mask=end)
            return c

        lax.fori_loop(0, NBLK, lbody, 0)

        pltpu.async_copy(acc, part_hbm.at[wid], so).wait()

    return k(data, ids)


def _phase2(part):
    def k(part_ref, out_ref):
        out_ref[...] = jnp.max(part_ref[...], axis=0)

    return pl.pallas_call(
        k,
        out_shape=jax.ShapeDtypeStruct((SPAD,), jnp.float32),
    )(part)


def kernel(data, segment_ids, num_segments):
    del num_segments
    ids = segment_ids.astype(jnp.int32)
    part = _phase1(data, ids)
    out = _phase2(part)
    return out[:S_SEG]


# speedup vs baseline: 2.7452x; 1.0015x over previous
import functools

import jax
import jax.numpy as jnp
from jax import lax
from jax.experimental import pallas as pl
from jax.experimental.pallas import tpu as pltpu
from jax.experimental.pallas import tpu_sc as plsc

N = 6_400_000
S_SEG = 10_000
L = 16
NW = 32
SPAD = 10_240
COLS = SPAD // NW
CHUNK = N // NW
BLK = 20_000
NBLK = CHUNK // BLK
SUBL = BLK // L
UNROLL = 5

_MESH = dict(core_axis_name="c", subcore_axis_name="s")
_PARAMS = pltpu.CompilerParams(
    needs_layout_passes=False, use_tc_tiling_on_sc=False
)


def _take(x, idx):
    return jnp.take_along_axis(x, idx, axis=0)


def _phase1(data, ids):
    mesh = plsc.VectorSubcoreMesh(**_MESH)

    @functools.partial(
        pl.kernel,
        out_type=jax.ShapeDtypeStruct((NW, SPAD), jnp.float32),
        mesh=mesh,
        scratch_types=[
            pltpu.VMEM((2, BLK), jnp.float32),
            pltpu.VMEM((2, BLK), jnp.int32),
            pltpu.VMEM((SPAD,), jnp.float32),
            pltpu.VMEM((NBLK * L,), jnp.int32),
            pltpu.VMEM((NBLK * L,), jnp.float32),
            pltpu.SemaphoreType.DMA,
            pltpu.SemaphoreType.DMA,
            pltpu.SemaphoreType.DMA,
            pltpu.SemaphoreType.DMA,
            pltpu.SemaphoreType.DMA,
        ],
        compiler_params=_PARAMS,
    )
    def k(data_hbm, ids_hbm, part_hbm, dbuf, ibuf, acc, lid, lval, sd0, sd1,
          si0, si1, so):
        wid = lax.axis_index("c") * 16 + lax.axis_index("s")
        base = wid * CHUNK
        dsem = (sd0, sd1)
        isem = (si0, si1)

        minf = jnp.full((L,), -jnp.inf, dtype=jnp.float32)
        iota = lax.iota(jnp.int32, L)
        last = jnp.full((L,), L - 1, dtype=jnp.int32)

        def ibody(i, c):
            acc[pl.ds(i * L, L)] = minf
            return c

        lax.fori_loop(0, SPAD // L, ibody, 0)

        def issue(b):
            slot = b % 2
            off = base + b * BLK
            pltpu.async_copy(data_hbm.at[pl.ds(off, BLK)], dbuf.at[slot],
                             dsem[slot])
            pltpu.async_copy(ids_hbm.at[pl.ds(off, BLK)], ibuf.at[slot],
                             isem[slot])

        def wait(b):
            slot = b % 2
            off = base + b * BLK
            pltpu.make_async_copy(data_hbm.at[pl.ds(off, BLK)], dbuf.at[slot],
                                  dsem[slot]).wait()
            pltpu.make_async_copy(ids_hbm.at[pl.ds(off, BLK)], ibuf.at[slot],
                                  isem[slot]).wait()

        lane_base = iota * SUBL

        def process(slot, b):
            prev0 = plsc.load_gather(ibuf.at[slot], [lane_base])

            @plsc.parallel_loop(0, SUBL, unroll=UNROLL,
                                carry=(prev0, minf))
            def step(t, carry):
                prev, accv = carry
                fidx = lane_base + t
                idv = plsc.load_gather(ibuf.at[slot], [fidx])
                dv = plsc.load_gather(dbuf.at[slot], [fidx])
                changed = idv != prev
                plsc.store_scatter(acc, [prev], accv, mask=changed)
                accv = jnp.where(changed, dv, jnp.maximum(accv, dv))
                return idv, accv

            prev, accv = step
            lid[pl.ds(b * L, L)] = prev
            lval[pl.ds(b * L, L)] = accv

        issue(0)
        for b in range(NBLK):
            if b + 1 < NBLK:
                issue(b + 1)
            wait(b)
            process(b % 2, b)

        def lbody(e, c):
            seg = lid[pl.ds(e * L, L)]
            vals = lval[pl.ds(e * L, L)]
            for sh in (1, 2, 4, 8):
                pidx = jnp.maximum(iota - sh, 0)
                gseg = _take(seg, pidx)
                gval = _take(vals, pidx)
                vals = jnp.where(seg == gseg, jnp.maximum(vals, gval), vals)
            nseg = _take(seg, jnp.minimum(iota + 1, last))
            end = (seg != nseg) | (iota == last)
            cur = plsc.load_gather(acc, [seg])
            plsc.store_scatter(acc, [seg], jnp.maximum(cur, vals), mask=end)
            return c

        lax.fori_loop(0, NBLK, lbody, 0)

        pltpu.async_copy(acc, part_hbm.at[wid], so).wait()

    return k(data, ids)


def _phase2(part):
    def k(part_ref, out_ref):
        out_ref[...] = jnp.max(part_ref[...], axis=0)

    return pl.pallas_call(
        k,
        out_shape=jax.ShapeDtypeStruct((SPAD,), jnp.float32),
    )(part)


def kernel(data, segment_ids, num_segments):
    del num_segments
    ids = segment_ids.astype(jnp.int32)
    part = _phase1(data, ids)
    out = _phase2(part)
    return out[:S_SEG]


# speedup vs baseline: 2.7886x; 1.0158x over previous
import functools

import jax
import jax.numpy as jnp
from jax import lax
from jax.experimental import pallas as pl
from jax.experimental.pallas import tpu as pltpu
from jax.experimental.pallas import tpu_sc as plsc

N = 6_400_000
S_SEG = 10_000
L = 16
NW = 32
SPAD = 10_240
COLS = SPAD // NW
CHUNK = N // NW
BLK = 20_000
SIZES = (4_000, 16_000) + (BLK,) * 9
OFFS = tuple(sum(SIZES[:i]) for i in range(len(SIZES)))
NBLK = len(SIZES)
UNROLL = 5

_MESH = dict(core_axis_name="c", subcore_axis_name="s")
_PARAMS = pltpu.CompilerParams(
    needs_layout_passes=False, use_tc_tiling_on_sc=False
)


def _take(x, idx):
    return jnp.take_along_axis(x, idx, axis=0)


def _phase1(data, ids):
    mesh = plsc.VectorSubcoreMesh(**_MESH)

    @functools.partial(
        pl.kernel,
        out_type=jax.ShapeDtypeStruct((NW, SPAD), jnp.float32),
        mesh=mesh,
        scratch_types=[
            pltpu.VMEM((2, BLK), jnp.float32),
            pltpu.VMEM((2, BLK), jnp.int32),
            pltpu.VMEM((SPAD,), jnp.float32),
            pltpu.VMEM((NBLK * L,), jnp.int32),
            pltpu.VMEM((NBLK * L,), jnp.float32),
            pltpu.SemaphoreType.DMA,
            pltpu.SemaphoreType.DMA,
            pltpu.SemaphoreType.DMA,
            pltpu.SemaphoreType.DMA,
            pltpu.SemaphoreType.DMA,
        ],
        compiler_params=_PARAMS,
    )
    def k(data_hbm, ids_hbm, part_hbm, dbuf, ibuf, acc, lid, lval, sd0, sd1,
          si0, si1, so):
        wid = lax.axis_index("c") * 16 + lax.axis_index("s")
        base = wid * CHUNK
        dsem = (sd0, sd1)
        isem = (si0, si1)

        minf = jnp.full((L,), -jnp.inf, dtype=jnp.float32)
        iota = lax.iota(jnp.int32, L)
        last = jnp.full((L,), L - 1, dtype=jnp.int32)

        def ibody(i, c):
            acc[pl.ds(i * L, L)] = minf
            return c

        lax.fori_loop(0, SPAD // L, ibody, 0)

        def issue(b):
            slot = b % 2
            off = base + OFFS[b]
            sz = SIZES[b]
            pltpu.async_copy(data_hbm.at[pl.ds(off, sz)],
                             dbuf.at[slot, pl.ds(0, sz)], dsem[slot])
            pltpu.async_copy(ids_hbm.at[pl.ds(off, sz)],
                             ibuf.at[slot, pl.ds(0, sz)], isem[slot])

        def wait(b):
            slot = b % 2
            off = base + OFFS[b]
            sz = SIZES[b]
            pltpu.make_async_copy(data_hbm.at[pl.ds(off, sz)],
                                  dbuf.at[slot, pl.ds(0, sz)],
                                  dsem[slot]).wait()
            pltpu.make_async_copy(ids_hbm.at[pl.ds(off, sz)],
                                  ibuf.at[slot, pl.ds(0, sz)],
                                  isem[slot]).wait()

        def process(slot, b):
            subl = SIZES[b] // L
            lane_base = iota * subl
            prev0 = plsc.load_gather(ibuf.at[slot], [lane_base])

            @plsc.parallel_loop(0, subl, unroll=UNROLL,
                                carry=(prev0, minf))
            def step(t, carry):
                prev, accv = carry
                fidx = lane_base + t
                idv = plsc.load_gather(ibuf.at[slot], [fidx])
                dv = plsc.load_gather(dbuf.at[slot], [fidx])
                changed = idv != prev
                plsc.store_scatter(acc, [prev], accv, mask=changed)
                accv = jnp.where(changed, dv, jnp.maximum(accv, dv))
                return idv, accv

            prev, accv = step
            lid[pl.ds(b * L, L)] = prev
            lval[pl.ds(b * L, L)] = accv

        issue(0)
        for b in range(NBLK):
            if b + 1 < NBLK:
                issue(b + 1)
            wait(b)
            process(b % 2, b)

        def lbody(e, c):
            seg = lid[pl.ds(e * L, L)]
            vals = lval[pl.ds(e * L, L)]
            for sh in (1, 2, 4, 8):
                pidx = jnp.maximum(iota - sh, 0)
                gseg = _take(seg, pidx)
                gval = _take(vals, pidx)
                vals = jnp.where(seg == gseg, jnp.maximum(vals, gval), vals)
            nseg = _take(seg, jnp.minimum(iota + 1, last))
            end = (seg != nseg) | (iota == last)
            cur = plsc.load_gather(acc, [seg])
            plsc.store_scatter(acc, [seg], jnp.maximum(cur, vals), mask=end)
            return c

        lax.fori_loop(0, NBLK, lbody, 0)

        pltpu.async_copy(acc, part_hbm.at[wid], so).wait()

    return k(data, ids)


def _phase2(part):
    def k(part_ref, out_ref):
        out_ref[...] = jnp.max(part_ref[...], axis=0)

    return pl.pallas_call(
        k,
        out_shape=jax.ShapeDtypeStruct((SPAD,), jnp.float32),
    )(part)


def kernel(data, segment_ids, num_segments):
    del num_segments
    ids = segment_ids.astype(jnp.int32)
    part = _phase1(data, ids)
    out = _phase2(part)
    return out[:S_SEG]
